# baseline (speedup 1.0000x reference)
import jax
import jax.numpy as jnp
import numpy as np
from jax import lax
from jax.experimental import pallas as pl
from jax.experimental.pallas import tpu as pltpu
from jax.experimental.pallas import tpu_sc as plsc

N = 20000
G = 20
NPAD = 20480
NW = 16
PW = NPAD // NW
NEG_THRESH = 0.3
POS_THRESH = 0.7
EPS = float(np.finfo(np.float32).eps)
LN2 = 0.6931471805599453
SQRT2 = 1.4142135


def _bcast_f(x):
    return jnp.full((16,), x, jnp.float32)


def _bcast_i(x):
    return jnp.full((16,), x, jnp.int32)


def _log16(x):
    bits = lax.bitcast_convert_type(x, jnp.int32)
    e = (bits >> 23) - 127
    m = lax.bitcast_convert_type((bits & 0x7FFFFF) | 0x3F800000, jnp.float32)
    big = m > SQRT2
    m = jnp.where(big, m * 0.5, m)
    e = jnp.where(big, e + 1, e)
    t = (m - 1.0) / (m + 1.0)
    t2 = t * t
    p = jnp.float32(2.0 / 9.0)
    p = p * t2 + jnp.float32(2.0 / 7.0)
    p = p * t2 + jnp.float32(2.0 / 5.0)
    p = p * t2 + jnp.float32(2.0 / 3.0)
    p = p * t2 + jnp.float32(2.0)
    return e.astype(jnp.float32) * jnp.float32(LN2) + t * p


def _sc_body(inp_h, bbox_h, out_h,
             stg, bbox_v, area_a, accmax, accarg, accfrc,
             gmax_v, gidx_v, allmax_v, allidx_v,
             parts_v, allparts_v, outv,
             smax, sidx, sparts):
    wid = lax.axis_index("s") + lax.axis_index("c") * NW
    base = wid * PW
    lane = lax.iota(jnp.int32, 16)

    pltpu.sync_copy(inp_h.at[:, pl.ds(base, PW)], stg)
    pltpu.sync_copy(bbox_h, bbox_v)

    @plsc.parallel_loop(0, PW, 16)
    def _(off):
        sl = pl.ds(off, 16)
        accmax[sl] = _bcast_f(-1.0)
        accarg[sl] = _bcast_i(0)
        accfrc[sl] = _bcast_i(0)
        area_a[sl] = (stg[2, sl] - stg[0, sl]) * (stg[3, sl] - stg[1, sl])

    KG = 2

    def g_body(gp, _):
        gs = [(gp * KG + k).astype(jnp.int32) for k in range(KG)]
        bb = [[plsc.load_gather(bbox_v, [_bcast_i(4 * g + c)])
               for c in range(4)] for g in gs]
        ab = [(b[2] - b[0]) * (b[3] - b[1]) for b in bb]

        init = tuple([v for _k in range(KG)
                      for v in (_bcast_f(-2.0), _bcast_i(NPAD))])

        @plsc.parallel_loop(0, PW, 16, unroll=4, carry=init)
        def cand(off, carry):
            sl = pl.ds(off, 16)
            x0 = stg[0, sl]; y0 = stg[1, sl]; x1 = stg[2, sl]; y1 = stg[3, sl]
            aa_v = area_a[sl]
            glob = base + off + lane
            am = accmax[sl]
            ag = accarg[sl]
            out = []
            for k in range(KG):
                b = bb[k]
                inter = (jnp.maximum(jnp.minimum(x1, b[2]) -
                                     jnp.maximum(x0, b[0]), 0.0) *
                         jnp.maximum(jnp.minimum(y1, b[3]) -
                                     jnp.maximum(y0, b[1]), 0.0))
                iou = inter / (aa_v + ab[k] - inter)
                upd = iou > am
                am = jnp.where(upd, iou, am)
                ag = jnp.where(upd, _bcast_i(gs[k]), ag)
                gm, gidx = carry[2 * k], carry[2 * k + 1]
                upd2 = iou > gm
                out.append(jnp.where(upd2, iou, gm))
                out.append(jnp.where(upd2, glob, gidx))
            accmax[sl] = am
            accarg[sl] = ag
            return tuple(out)

        onelane = lane == 0
        for k in range(KG):
            gmaxv, gidxv = cand[2 * k], cand[2 * k + 1]
            m = jnp.max(gmaxv)
            mi = jnp.min(jnp.where(gmaxv == m, gidxv, _bcast_i(NPAD)))
            plsc.store_scatter(gmax_v, [_bcast_i(gs[k])], _bcast_f(m),
                               mask=onelane)
            plsc.store_scatter(gidx_v, [_bcast_i(gs[k])], _bcast_i(mi),
                               mask=onelane)
        return 0

    lax.fori_loop(0, G // KG, g_body, 0)

    pltpu.sync_copy(gmax_v, smax.at[pl.ds(wid * 32, 32)])
    pltpu.sync_copy(gidx_v, sidx.at[pl.ds(wid * 32, 32)])
    plsc.subcore_barrier()
    pltpu.sync_copy(smax, allmax_v)
    pltpu.sync_copy(sidx, allidx_v)

    onelane = lane == 0
    for g in range(G):
        gcol = _bcast_i(g)
        fidx = lane * 32 + gcol
        vals = plsc.load_gather(allmax_v, [fidx])
        idxs = plsc.load_gather(allidx_v, [fidx])
        m = jnp.max(vals)
        mi = jnp.min(jnp.where(vals == m, idxs, _bcast_i(NPAD)))
        loc = mi - base
        inr = (loc >= 0) & (loc < PW)
        lc = _bcast_i(jnp.clip(loc, 0, PW - 1))
        msk = onelane & inr
        plsc.store_scatter(accarg, [lc], gcol, mask=msk)
        plsc.store_scatter(accfrc, [lc], _bcast_i(1), mask=msk)

    z = _bcast_f(0.0)

    @plsc.parallel_loop(0, PW, 16, carry=(z, z, z, z, z))
    def sums(off, carry):
        sp, sv, slc, sce, sse = carry
        sl = pl.ds(off, 16)
        am = accmax[sl]
        aa = accarg[sl]
        af = accfrc[sl]
        glob = base + off + lane
        real = glob < N
        pos = ((am >= POS_THRESH) | (af == 1)) & real
        neg = (am < NEG_THRESH) & (af == 0) & real
        valid = pos | neg
        posf = jnp.where(pos, 1.0, 0.0).astype(jnp.float32)
        validf = jnp.where(valid, 1.0, 0.0).astype(jnp.float32)

        x0 = stg[0, sl]; y0 = stg[1, sl]; x1 = stg[2, sl]; y1 = stg[3, sl]
        b4 = aa * 4
        sx0 = plsc.load_gather(bbox_v, [b4])
        sy0 = plsc.load_gather(bbox_v, [b4 + 1])
        sx1 = plsc.load_gather(bbox_v, [b4 + 2])
        sy1 = plsc.load_gather(bbox_v, [b4 + 3])

        w = jnp.maximum(x1 - x0, EPS)
        h = jnp.maximum(y1 - y0, EPS)
        cx = x0 + 0.5 * (x1 - x0)
        cy = y0 + 0.5 * (y1 - y0)
        bw = sx1 - sx0
        bh = sy1 - sy0
        bcx = sx0 + 0.5 * bw
        bcy = sy0 + 0.5 * bh
        d0 = (bcx - cx) / w
        d1 = (bcy - cy) / h
        d2 = _log16(bw / w)
        d3 = _log16(bh / h)

        lsum = _bcast_f(0.0)
        for dv, j in ((d0, 4), (d1, 5), (d2, 6), (d3, 7)):
            ad = jnp.abs(dv - stg[j, sl])
            lsum = lsum + jnp.where(ad < 1.0, 0.5 * ad * ad, ad - 0.5)

        a0 = stg[8, sl]
        a1 = stg[9, sl]
        mx = jnp.maximum(a0, a1)
        lz = mx + _log16(1.0 + jnp.exp(jnp.minimum(a0, a1) - mx))
        ce = lz - jnp.where(pos, a1, a0)

        fgd = stg[10, sl] - am
        se = fgd * fgd
        return (sp + posf, sv + validf, slc + lsum * posf,
                sce + ce * validf, sse + se * posf)

    sp, sv, slc, sce, sse = sums

    pv = jnp.where(lane == 0, _bcast_f(jnp.sum(sp)),
         jnp.where(lane == 1, _bcast_f(jnp.sum(sv)),
         jnp.where(lane == 2, _bcast_f(jnp.sum(slc)),
         jnp.where(lane == 3, _bcast_f(jnp.sum(sce)),
         jnp.where(lane == 4, _bcast_f(jnp.sum(sse)), _bcast_f(0.0))))))
    parts_v[...] = pv
    pltpu.sync_copy(parts_v, sparts.at[pl.ds(wid * 16, 16)])
    plsc.subcore_barrier()

    @pl.when(wid == 0)
    def _():
        pltpu.sync_copy(sparts, allparts_v)
        tot_p = jnp.sum(plsc.load_gather(allparts_v, [lane * 16 + _bcast_i(0)]))
        tot_v = jnp.sum(plsc.load_gather(allparts_v, [lane * 16 + _bcast_i(1)]))
        tot_l = jnp.sum(plsc.load_gather(allparts_v, [lane * 16 + _bcast_i(2)]))
        tot_c = jnp.sum(plsc.load_gather(allparts_v, [lane * 16 + _bcast_i(3)]))
        tot_s = jnp.sum(plsc.load_gather(allparts_v, [lane * 16 + _bcast_i(4)]))
        npos = jnp.maximum(_bcast_f(tot_p), 1.0)
        nval = jnp.maximum(_bcast_f(tot_v), 1.0)
        loc_l = _bcast_f(tot_l) / npos
        cls_l = _bcast_f(tot_c) / nval
        reg_l = _bcast_f(tot_s) / npos
        outvec = jnp.where(lane == 0, loc_l,
                 jnp.where(lane == 1, cls_l,
                 jnp.where(lane == 2, reg_l,
                 loc_l + cls_l + reg_l)))
        outv[...] = outvec
        pltpu.sync_copy(outv, out_h)


_vm = lambda shp, dt: pltpu.VMEM(shp, dt)
_sc_call = pl.kernel(
    _sc_body,
    out_type=jax.ShapeDtypeStruct((16,), jnp.float32),
    mesh=plsc.VectorSubcoreMesh(core_axis_name="c", subcore_axis_name="s",
                                num_cores=1),
    scratch_types=[
        _vm((11, PW), jnp.float32),
        _vm((96,), jnp.float32),
        _vm((PW,), jnp.float32),
        _vm((PW,), jnp.float32), _vm((PW,), jnp.int32), _vm((PW,), jnp.int32),
        _vm((32,), jnp.float32), _vm((32,), jnp.int32),
        _vm((NW * 32,), jnp.float32), _vm((NW * 32,), jnp.int32),
        _vm((16,), jnp.float32), _vm((NW * 16,), jnp.float32),
        _vm((16,), jnp.float32),
        pltpu.VMEM_SHARED((NW * 32,), jnp.float32),
        pltpu.VMEM_SHARED((NW * 32,), jnp.int32),
        pltpu.VMEM_SHARED((NW * 16,), jnp.float32),
    ],
    compiler_params=pltpu.CompilerParams(needs_layout_passes=False),
)


def kernel(anchor, bbox, gt_label, rpn_loc, rpn_score, rpn_fg_score):
    allc = jnp.concatenate(
        [anchor, rpn_loc, rpn_score, rpn_fg_score[:, None]], axis=1)
    inp = jnp.pad(allc, ((0, NPAD - N), (0, 0))).T
    bboxf = jnp.pad(bbox.reshape(-1), (0, 16))
    out = _sc_call(inp, bboxf)
    return (out[0], out[1], out[2], out[3])


# speedup vs baseline: 1.0104x; 1.0104x over previous
import jax
import jax.numpy as jnp
import numpy as np
from jax import lax
from jax.experimental import pallas as pl
from jax.experimental.pallas import tpu as pltpu
from jax.experimental.pallas import tpu_sc as plsc

N = 20000
G = 20
NPAD = 20480
NW = 16
PW = NPAD // NW
NEG_THRESH = 0.3
POS_THRESH = 0.7
EPS = float(np.finfo(np.float32).eps)
LN2 = 0.6931471805599453
SQRT2 = 1.4142135


def _bcast_f(x):
    return jnp.full((16,), x, jnp.float32)


def _bcast_i(x):
    return jnp.full((16,), x, jnp.int32)


def _log16(x):
    bits = lax.bitcast_convert_type(x, jnp.int32)
    e = (bits >> 23) - 127
    m = lax.bitcast_convert_type((bits & 0x7FFFFF) | 0x3F800000, jnp.float32)
    big = m > SQRT2
    m = jnp.where(big, m * 0.5, m)
    e = jnp.where(big, e + 1, e)
    t = (m - 1.0) / (m + 1.0)
    t2 = t * t
    p = jnp.float32(2.0 / 9.0)
    p = p * t2 + jnp.float32(2.0 / 7.0)
    p = p * t2 + jnp.float32(2.0 / 5.0)
    p = p * t2 + jnp.float32(2.0 / 3.0)
    p = p * t2 + jnp.float32(2.0)
    return e.astype(jnp.float32) * jnp.float32(LN2) + t * p


def _sc_body(inp_h, bbox_h, out_h,
             stg, bbox_v, area_a, accmax, accarg, accfrc,
             gmax_v, gidx_v, allmax_v, allidx_v,
             parts_v, allparts_v, outv,
             smax, sidx, sparts):
    wid = lax.axis_index("s") + lax.axis_index("c") * NW
    base = wid * PW
    lane = lax.iota(jnp.int32, 16)

    pltpu.sync_copy(inp_h.at[:, pl.ds(base, PW)], stg)
    pltpu.sync_copy(bbox_h, bbox_v)

    @plsc.parallel_loop(0, PW, 16)
    def _(off):
        sl = pl.ds(off, 16)
        accmax[sl] = _bcast_f(-1.0)
        accarg[sl] = _bcast_i(0)
        accfrc[sl] = _bcast_i(0)
        area_a[sl] = (stg[2, sl] - stg[0, sl]) * (stg[3, sl] - stg[1, sl])

    KG = 2

    def g_body(gp, _):
        gs = [(gp * KG + k).astype(jnp.int32) for k in range(KG)]
        bb = [[plsc.load_gather(bbox_v, [_bcast_i(4 * g + c)])
               for c in range(4)] for g in gs]
        ab = [(b[2] - b[0]) * (b[3] - b[1]) for b in bb]

        init = tuple([v for _k in range(KG)
                      for v in (_bcast_f(-2.0), _bcast_i(NPAD))])

        @plsc.parallel_loop(0, PW, 16, unroll=2, carry=init)
        def cand(off, carry):
            sl = pl.ds(off, 16)
            x0 = stg[0, sl]; y0 = stg[1, sl]; x1 = stg[2, sl]; y1 = stg[3, sl]
            aa_v = area_a[sl]
            glob = base + off + lane
            am = accmax[sl]
            ag = accarg[sl]
            out = []
            for k in range(KG):
                b = bb[k]
                inter = (jnp.maximum(jnp.minimum(x1, b[2]) -
                                     jnp.maximum(x0, b[0]), 0.0) *
                         jnp.maximum(jnp.minimum(y1, b[3]) -
                                     jnp.maximum(y0, b[1]), 0.0))
                iou = inter / (aa_v + ab[k] - inter)
                upd = iou > am
                am = jnp.where(upd, iou, am)
                ag = jnp.where(upd, _bcast_i(gs[k]), ag)
                gm, gidx = carry[2 * k], carry[2 * k + 1]
                upd2 = iou > gm
                out.append(jnp.where(upd2, iou, gm))
                out.append(jnp.where(upd2, glob, gidx))
            accmax[sl] = am
            accarg[sl] = ag
            return tuple(out)

        onelane = lane == 0
        for k in range(KG):
            gmaxv, gidxv = cand[2 * k], cand[2 * k + 1]
            m = jnp.max(gmaxv)
            mi = jnp.min(jnp.where(gmaxv == m, gidxv, _bcast_i(NPAD)))
            plsc.store_scatter(gmax_v, [_bcast_i(gs[k])], _bcast_f(m),
                               mask=onelane)
            plsc.store_scatter(gidx_v, [_bcast_i(gs[k])], _bcast_i(mi),
                               mask=onelane)
        return 0

    lax.fori_loop(0, G // KG, g_body, 0)

    pltpu.sync_copy(gmax_v, smax.at[pl.ds(wid * 32, 32)])
    pltpu.sync_copy(gidx_v, sidx.at[pl.ds(wid * 32, 32)])
    plsc.subcore_barrier()
    pltpu.sync_copy(smax, allmax_v)
    pltpu.sync_copy(sidx, allidx_v)

    onelane = lane == 0
    for g in range(G):
        gcol = _bcast_i(g)
        fidx = lane * 32 + gcol
        vals = plsc.load_gather(allmax_v, [fidx])
        idxs = plsc.load_gather(allidx_v, [fidx])
        m = jnp.max(vals)
        mi = jnp.min(jnp.where(vals == m, idxs, _bcast_i(NPAD)))
        loc = mi - base
        inr = (loc >= 0) & (loc < PW)
        lc = _bcast_i(jnp.clip(loc, 0, PW - 1))
        msk = onelane & inr
        plsc.store_scatter(accarg, [lc], gcol, mask=msk)
        plsc.store_scatter(accfrc, [lc], _bcast_i(1), mask=msk)

    z = _bcast_f(0.0)

    @plsc.parallel_loop(0, PW, 16, carry=(z, z, z, z, z))
    def sums(off, carry):
        sp, sv, slc, sce, sse = carry
        sl = pl.ds(off, 16)
        am = accmax[sl]
        aa = accarg[sl]
        af = accfrc[sl]
        glob = base + off + lane
        real = glob < N
        pos = ((am >= POS_THRESH) | (af == 1)) & real
        neg = (am < NEG_THRESH) & (af == 0) & real
        valid = pos | neg
        posf = jnp.where(pos, 1.0, 0.0).astype(jnp.float32)
        validf = jnp.where(valid, 1.0, 0.0).astype(jnp.float32)

        x0 = stg[0, sl]; y0 = stg[1, sl]; x1 = stg[2, sl]; y1 = stg[3, sl]
        b4 = aa * 4
        sx0 = plsc.load_gather(bbox_v, [b4])
        sy0 = plsc.load_gather(bbox_v, [b4 + 1])
        sx1 = plsc.load_gather(bbox_v, [b4 + 2])
        sy1 = plsc.load_gather(bbox_v, [b4 + 3])

        w = jnp.maximum(x1 - x0, EPS)
        h = jnp.maximum(y1 - y0, EPS)
        cx = x0 + 0.5 * (x1 - x0)
        cy = y0 + 0.5 * (y1 - y0)
        bw = sx1 - sx0
        bh = sy1 - sy0
        bcx = sx0 + 0.5 * bw
        bcy = sy0 + 0.5 * bh
        d0 = (bcx - cx) / w
        d1 = (bcy - cy) / h
        d2 = _log16(bw / w)
        d3 = _log16(bh / h)

        lsum = _bcast_f(0.0)
        for dv, j in ((d0, 4), (d1, 5), (d2, 6), (d3, 7)):
            ad = jnp.abs(dv - stg[j, sl])
            lsum = lsum + jnp.where(ad < 1.0, 0.5 * ad * ad, ad - 0.5)

        a0 = stg[8, sl]
        a1 = stg[9, sl]
        mx = jnp.maximum(a0, a1)
        lz = mx + _log16(1.0 + jnp.exp(jnp.minimum(a0, a1) - mx))
        ce = lz - jnp.where(pos, a1, a0)

        fgd = stg[10, sl] - am
        se = fgd * fgd
        return (sp + posf, sv + validf, slc + lsum * posf,
                sce + ce * validf, sse + se * posf)

    sp, sv, slc, sce, sse = sums

    pv = jnp.where(lane == 0, _bcast_f(jnp.sum(sp)),
         jnp.where(lane == 1, _bcast_f(jnp.sum(sv)),
         jnp.where(lane == 2, _bcast_f(jnp.sum(slc)),
         jnp.where(lane == 3, _bcast_f(jnp.sum(sce)),
         jnp.where(lane == 4, _bcast_f(jnp.sum(sse)), _bcast_f(0.0))))))
    parts_v[...] = pv
    pltpu.sync_copy(parts_v, sparts.at[pl.ds(wid * 16, 16)])
    plsc.subcore_barrier()

    @pl.when(wid == 0)
    def _():
        pltpu.sync_copy(sparts, allparts_v)
        tot_p = jnp.sum(plsc.load_gather(allparts_v, [lane * 16 + _bcast_i(0)]))
        tot_v = jnp.sum(plsc.load_gather(allparts_v, [lane * 16 + _bcast_i(1)]))
        tot_l = jnp.sum(plsc.load_gather(allparts_v, [lane * 16 + _bcast_i(2)]))
        tot_c = jnp.sum(plsc.load_gather(allparts_v, [lane * 16 + _bcast_i(3)]))
        tot_s = jnp.sum(plsc.load_gather(allparts_v, [lane * 16 + _bcast_i(4)]))
        npos = jnp.maximum(_bcast_f(tot_p), 1.0)
        nval = jnp.maximum(_bcast_f(tot_v), 1.0)
        loc_l = _bcast_f(tot_l) / npos
        cls_l = _bcast_f(tot_c) / nval
        reg_l = _bcast_f(tot_s) / npos
        outvec = jnp.where(lane == 0, loc_l,
                 jnp.where(lane == 1, cls_l,
                 jnp.where(lane == 2, reg_l,
                 loc_l + cls_l + reg_l)))
        outv[...] = outvec
        pltpu.sync_copy(outv, out_h)


_vm = lambda shp, dt: pltpu.VMEM(shp, dt)
_sc_call = pl.kernel(
    _sc_body,
    out_type=jax.ShapeDtypeStruct((16,), jnp.float32),
    mesh=plsc.VectorSubcoreMesh(core_axis_name="c", subcore_axis_name="s",
                                num_cores=1),
    scratch_types=[
        _vm((11, PW), jnp.float32),
        _vm((96,), jnp.float32),
        _vm((PW,), jnp.float32),
        _vm((PW,), jnp.float32), _vm((PW,), jnp.int32), _vm((PW,), jnp.int32),
        _vm((32,), jnp.float32), _vm((32,), jnp.int32),
        _vm((NW * 32,), jnp.float32), _vm((NW * 32,), jnp.int32),
        _vm((16,), jnp.float32), _vm((NW * 16,), jnp.float32),
        _vm((16,), jnp.float32),
        pltpu.VMEM_SHARED((NW * 32,), jnp.float32),
        pltpu.VMEM_SHARED((NW * 32,), jnp.int32),
        pltpu.VMEM_SHARED((NW * 16,), jnp.float32),
    ],
    compiler_params=pltpu.CompilerParams(needs_layout_passes=False),
)


def kernel(anchor, bbox, gt_label, rpn_loc, rpn_score, rpn_fg_score):
    allc = jnp.concatenate(
        [anchor, rpn_loc, rpn_score, rpn_fg_score[:, None]], axis=1)
    inp = jnp.pad(allc, ((0, NPAD - N), (0, 0))).T
    bboxf = jnp.pad(bbox.reshape(-1), (0, 16))
    out = _sc_call(inp, bboxf)
    return (out[0], out[1], out[2], out[3])
